# baseline (speedup 1.0000x reference)
import functools

import jax
import jax.numpy as jnp
from jax import lax
from jax.experimental import pallas as pl
from jax.experimental.pallas import tpu as pltpu
from jax.experimental.pallas import tpu_sc as plsc

NC = 2
NS = 16
NW = NC * NS
L = 16


def _choose_chunk(spt: int) -> int:
    c = 16
    for cand in (128, 112, 96, 80, 64, 48, 32, 16):
        if spt % cand == 0:
            c = cand
            break
    return c


def _make_kernel(B, F, FEAT_NUM, D):
    assert D == L and B % (NW * 16) == 0
    SPT = B // NW
    C = _choose_chunk(SPT)
    NCHUNK = SPT // C
    CF = C * F
    assert CF % L == 0
    PERIOD = 208
    assert CF % PERIOD == 0
    NSTREAM = CF // 128 if CF % 128 == 0 else None
    assert NSTREAM is not None

    mesh = plsc.VectorSubcoreMesh(
        core_axis_name="c", subcore_axis_name="s",
        num_cores=NC, num_subcores=NS)

    @functools.partial(
        pl.kernel,
        out_type=jax.ShapeDtypeStruct((B,), jnp.float32),
        mesh=mesh,
        compiler_params=pltpu.CompilerParams(
            needs_layout_passes=False, use_tc_tiling_on_sc=False),
        scratch_types=dict(
            idx_a=pltpu.VMEM((CF,), jnp.int32),
            idx_b=pltpu.VMEM((CF,), jnp.int32),
            rows_a=pltpu.VMEM((CF, L), jnp.float32),
            rows_b=pltpu.VMEM((CF, L), jnp.float32),
            wv_a=pltpu.VMEM((CF,), jnp.float32),
            wv_b=pltpu.VMEM((CF,), jnp.float32),
            ptf=pltpu.VMEM((L * C,), jnp.float32),
            out_v=pltpu.VMEM((C,), jnp.float32),
            offs_v=pltpu.VMEM((PERIOD,), jnp.int32),
            w0_v=pltpu.VMEM((L,), jnp.float32),
            sem_a=pltpu.SemaphoreType.DMA,
            sem_b=pltpu.SemaphoreType.DMA,
        ),
    )
    def fm_kernel(inp_ref, w0_ref, w_ref, v_ref, offs_ref, out_ref, *,
                  idx_a, idx_b, rows_a, rows_b, wv_a, wv_b,
                  ptf, out_v, offs_v, w0_v, sem_a, sem_b):
        wid = lax.axis_index("s") * NC + lax.axis_index("c")
        tile_base = wid * SPT

        pltpu.sync_copy(offs_ref, offs_v)
        pltpu.sync_copy(w0_ref, w0_v)

        iota = lax.iota(jnp.int32, L)
        zeros16 = jnp.zeros((L,), jnp.int32)
        offs_regs = [offs_v[pl.ds(L * j, L)] for j in range(PERIOD // L)]

        bufs = [(idx_a, rows_a, wv_a, sem_a), (idx_b, rows_b, wv_b, sem_b)]

        def stage(n, buf):
            idx, rows, wv, sem = buf
            fbase = (tile_base + n * C) * F
            pltpu.sync_copy(inp_ref.at[pl.ds(fbase, CF)], idx)

            def blk(b, carry):
                base = b * PERIOD
                for j in range(PERIOD // L):
                    sl = pl.ds(base + L * j, L)
                    idx[sl] = idx[sl] + offs_regs[j]
                return carry
            lax.fori_loop(0, CF // PERIOD, blk, 0)

            handles = []
            for r in range(NSTREAM):
                isl = idx.at[pl.ds(128 * r, 128)]
                handles.append(pltpu.async_copy(
                    v_ref.at[isl], rows.at[pl.ds(128 * r, 128)], sem))
                handles.append(pltpu.async_copy(
                    w_ref.at[isl], wv.at[pl.ds(128 * r, 128)], sem))
            return handles

        def compute(n, buf, handles):
            idx, rows, wv, sem = buf
            for h in handles:
                h.wait()

            def sbody(s, carry):
                p0 = F * s
                r0 = rows[p0]
                acc = r0
                ssq = r0 * r0
                for f in range(1, F):
                    rr = rows[p0 + f]
                    acc = acc + rr
                    ssq = ssq + rr * rr
                partial = acc * acc - ssq
                plsc.store_scatter(ptf, [iota * C + s], partial)
                return carry
            lax.fori_loop(0, C, sbody, 0)

            w0vec = w0_v[...]
            i_f = iota * F

            def gbody(g, carry):
                gb = L * F * g
                wacc = plsc.load_gather(wv, [i_f + gb])
                for f in range(1, F):
                    wacc = wacc + plsc.load_gather(wv, [i_f + (gb + f)])
                cs = ptf[pl.ds(L * g, L)]
                for d in range(1, L):
                    cs = cs + ptf[pl.ds(d * C + L * g, L)]
                out_v[pl.ds(L * g, L)] = w0vec + wacc + 0.5 * cs
                return carry
            lax.fori_loop(0, C // L, gbody, 0)

            pltpu.sync_copy(out_v, out_ref.at[pl.ds(tile_base + n * C, C)])

        handles = stage(0, bufs[0])
        for n in range(NCHUNK):
            nxt = None
            if n + 1 < NCHUNK:
                nxt = stage(n + 1, bufs[(n + 1) % 2])
            compute(n, bufs[n % 2], handles)
            handles = nxt

    return fm_kernel


def kernel(inputs, w0, w, v):
    B, F = inputs.shape
    D = v.shape[1]
    FEAT_NUM = w.shape[0] // F
    fm = _make_kernel(B, F, FEAT_NUM, D)
    inputs_flat = inputs.reshape(B * F)
    offs = (jnp.arange(208, dtype=jnp.int32) % F) * FEAT_NUM
    w0p = jnp.broadcast_to(w0.reshape(1), (L,)).astype(jnp.float32)
    out = fm(inputs_flat, w0p, w.reshape(-1), v, offs)
    return out.reshape(B, 1)
